# baseline (speedup 1.0000x reference)
import dataclasses
import functools

import jax
import jax.numpy as jnp
from jax import lax
from jax.experimental import pallas as pl
from jax.experimental.pallas import tpu as pltpu
from jax.experimental.pallas import tpu_sc as plsc

EMBED = 64
LINE = 2 * EMBED
VOCAB = 1000000
RBLK = 512
NPACK = (VOCAB + RBLK - 1) // RBLK
NLINES = NPACK * (RBLK // 2)
LANES = 16
NCORE = 2
NSUB = 16
NWORK = NCORE * NSUB
BATCH = 16384
BPW = BATCH // NWORK
PCHUNK = 128
NCH = BPW // PCHUNK
GROUP = LANES
KCH = EMBED // LANES

_cp = pltpu.CompilerParams(use_tc_tiling_on_sc=False)
if "needs_layout_passes" in pltpu.CompilerParams.__dataclass_fields__:
    _cp = dataclasses.replace(_cp, needs_layout_passes=False)


def _pack_body(ct_ref, wt_ref, cp_ref, wp_ref):
    eye = jax.lax.broadcasted_iota(jnp.int32, (EMBED, EMBED), 0)
    eyet = jax.lax.broadcasted_iota(jnp.int32, (EMBED, EMBED), 1)
    ident = jnp.where(eye == eyet, 1.0, 0.0).astype(jnp.float32)

    def tr(x):
        return jax.lax.dot_general(
            x, ident, (((0,), (0,)), ((), ())),
            preferred_element_type=jnp.float32)

    cp_ref[:, 0:EMBED] = tr(ct_ref[:, 0:RBLK // 2])
    cp_ref[:, EMBED:LINE] = tr(ct_ref[:, RBLK // 2:RBLK])
    wp_ref[:, 0:EMBED] = tr(wt_ref[:, 0:RBLK // 2])
    wp_ref[:, EMBED:LINE] = tr(wt_ref[:, RBLK // 2:RBLK])


def _pack(ctT, wtT):
    return pl.pallas_call(
        _pack_body,
        grid=(NPACK,),
        in_specs=[
            pl.BlockSpec((EMBED, RBLK), lambda q: (0, q)),
            pl.BlockSpec((EMBED, RBLK), lambda q: (0, q)),
        ],
        out_specs=[
            pl.BlockSpec((RBLK // 2, LINE), lambda q: (q, 0)),
            pl.BlockSpec((RBLK // 2, LINE), lambda q: (q, 0)),
        ],
        out_shape=[
            jax.ShapeDtypeStruct((NLINES, LINE), jnp.float32),
            jax.ShapeDtypeStruct((NLINES, LINE), jnp.float32),
        ],
        compiler_params=pltpu.CompilerParams(
            dimension_semantics=("parallel",)),
    )(ctT, wtT)


@functools.partial(
    pl.kernel,
    compiler_params=_cp,
    out_type=jax.ShapeDtypeStruct((BATCH,), jnp.float32),
    mesh=plsc.VectorSubcoreMesh(core_axis_name="c", subcore_axis_name="s"),
    scratch_types=[
        pltpu.VMEM((BPW,), jnp.int32),
        pltpu.VMEM((BPW,), jnp.int32),
        pltpu.VMEM((BPW,), jnp.int32),
        pltpu.VMEM((BPW,), jnp.int32),
        pltpu.VMEM((PCHUNK, LINE), jnp.float32),
        pltpu.VMEM((PCHUNK, LINE), jnp.float32),
        pltpu.VMEM((GROUP, LANES), jnp.float32),
        pltpu.VMEM((BPW,), jnp.float32),
        pltpu.SemaphoreType.DMA,
    ],
)
def _sgns_sc(c_hbm, w_hbm, cpack_hbm, wpack_hbm, out_hbm,
             cline, wline, chalf, whalf, cbuf, wbuf, accbuf, outv, sem):
    wid = lax.axis_index("s") * NCORE + lax.axis_index("c")
    base = wid * BPW

    pltpu.sync_copy(c_hbm.at[pl.ds(base, BPW)], cline)
    pltpu.sync_copy(w_hbm.at[pl.ds(base, BPW)], wline)
    m255 = jnp.full((LANES,), 255, jnp.int32)
    half_lim = jnp.full((LANES,), 255, jnp.int32)
    for v in range(BPW // LANES):
        sl = pl.ds(v * LANES, LANES)
        cv = cline[sl]
        wv = wline[sl]
        chalf[sl] = (cv >> 8) & jnp.full((LANES,), 1, jnp.int32)
        whalf[sl] = (wv >> 8) & jnp.full((LANES,), 1, jnp.int32)
        cline[sl] = ((cv >> 9) << 8) + (cv & m255)
        wline[sl] = ((wv >> 9) << 8) + (wv & m255)
    del half_lim

    row_iota = lax.iota(jnp.int32, LANES)
    fone = jnp.full((LANES,), 1.0, jnp.float32)
    izero = jnp.zeros((LANES,), jnp.int32)

    @pl.loop(0, NCH)
    def _(ch):
        p0 = ch * PCHUNK
        cg = pltpu.async_copy(
            cpack_hbm.at[cline.at[pl.ds(p0, PCHUNK)]], cbuf, sem)
        wg = pltpu.async_copy(
            wpack_hbm.at[wline.at[pl.ds(p0, PCHUNK)]], wbuf, sem)
        cg.wait()
        wg.wait()

        for g in range(PCHUNK // GROUP):
            for r in range(GROUP):
                p = g * GROUP + r
                psplat = jnp.full((LANES,), p0 + p, jnp.int32)
                cmask = plsc.load_gather(chalf, [psplat]) > izero
                wmask = plsc.load_gather(whalf, [psplat]) > izero
                acc = None
                for k in range(KCH):
                    clo = cbuf[p, pl.ds(k * LANES, LANES)]
                    chi = cbuf[p, pl.ds(EMBED + k * LANES, LANES)]
                    wlo = wbuf[p, pl.ds(k * LANES, LANES)]
                    whi = wbuf[p, pl.ds(EMBED + k * LANES, LANES)]
                    cv = jnp.where(cmask, chi, clo)
                    wv = jnp.where(wmask, whi, wlo)
                    prod = cv * wv
                    acc = prod if acc is None else acc + prod
                accbuf[r, :] = acc
            tot = None
            for j in range(LANES):
                col = plsc.load_gather(
                    accbuf, [row_iota, jnp.full((LANES,), j, jnp.int32)])
                tot = col if tot is None else tot + col
            outv[pl.ds(p0 + g * GROUP, GROUP)] = fone / (fone + jnp.exp(-tot))

    pltpu.sync_copy(outv, out_hbm.at[pl.ds(base, BPW)])


def kernel(c, w, c_table, w_table):
    cpack, wpack = _pack(c_table.T, w_table.T)
    return _sgns_sc(c, w, cpack, wpack)


# speedup vs baseline: 2.7055x; 2.7055x over previous
import dataclasses
import functools

import jax
import jax.numpy as jnp
from jax import lax
from jax.experimental import pallas as pl
from jax.experimental.pallas import tpu as pltpu
from jax.experimental.pallas import tpu_sc as plsc

EMBED = 64
LINE = 2 * EMBED
VOCAB = 1000000
RBLK = 4096
NPACK = (VOCAB + RBLK - 1) // RBLK
NLINES = NPACK * (RBLK // 2)
LANES = 16
NCORE = 2
NSUB = 16
NWORK = NCORE * NSUB
BATCH = 16384
BPW = BATCH // NWORK
PCHUNK = 128
NCH = BPW // PCHUNK
GROUP = LANES
KCH = EMBED // LANES

_cp = pltpu.CompilerParams(use_tc_tiling_on_sc=False)
if "needs_layout_passes" in pltpu.CompilerParams.__dataclass_fields__:
    _cp = dataclasses.replace(_cp, needs_layout_passes=False)


def _pack_body(ct_ref, wt_ref, cp_ref, wp_ref):
    cp_ref[:, 0:EMBED] = ct_ref[:, 0:RBLK // 2].T
    cp_ref[:, EMBED:LINE] = ct_ref[:, RBLK // 2:RBLK].T
    wp_ref[:, 0:EMBED] = wt_ref[:, 0:RBLK // 2].T
    wp_ref[:, EMBED:LINE] = wt_ref[:, RBLK // 2:RBLK].T


def _pack(ctT, wtT):
    return pl.pallas_call(
        _pack_body,
        grid=(NPACK,),
        in_specs=[
            pl.BlockSpec((EMBED, RBLK), lambda q: (0, q)),
            pl.BlockSpec((EMBED, RBLK), lambda q: (0, q)),
        ],
        out_specs=[
            pl.BlockSpec((RBLK // 2, LINE), lambda q: (q, 0)),
            pl.BlockSpec((RBLK // 2, LINE), lambda q: (q, 0)),
        ],
        out_shape=[
            jax.ShapeDtypeStruct((NLINES, LINE), jnp.float32),
            jax.ShapeDtypeStruct((NLINES, LINE), jnp.float32),
        ],
        compiler_params=pltpu.CompilerParams(
            dimension_semantics=("parallel",)),
    )(ctT, wtT)


@functools.partial(
    pl.kernel,
    compiler_params=_cp,
    out_type=jax.ShapeDtypeStruct((BATCH,), jnp.float32),
    mesh=plsc.VectorSubcoreMesh(core_axis_name="c", subcore_axis_name="s"),
    scratch_types=[
        pltpu.VMEM((BPW,), jnp.int32),
        pltpu.VMEM((BPW,), jnp.int32),
        pltpu.VMEM((BPW,), jnp.int32),
        pltpu.VMEM((BPW,), jnp.int32),
        pltpu.VMEM((PCHUNK, LINE), jnp.float32),
        pltpu.VMEM((PCHUNK, LINE), jnp.float32),
        pltpu.VMEM((GROUP, LANES), jnp.float32),
        pltpu.VMEM((BPW,), jnp.float32),
        pltpu.SemaphoreType.DMA,
    ],
)
def _sgns_sc(c_hbm, w_hbm, cpack_hbm, wpack_hbm, out_hbm,
             cline, wline, chalf, whalf, cbuf, wbuf, accbuf, outv, sem):
    wid = lax.axis_index("s") * NCORE + lax.axis_index("c")
    base = wid * BPW

    pltpu.sync_copy(c_hbm.at[pl.ds(base, BPW)], cline)
    pltpu.sync_copy(w_hbm.at[pl.ds(base, BPW)], wline)
    lmask = jnp.full((LANES,), RBLK // 2 - 1, jnp.int32)
    ibit = jnp.full((LANES,), 1, jnp.int32)
    hsh = (RBLK // 2).bit_length() - 1
    for v in range(BPW // LANES):
        sl = pl.ds(v * LANES, LANES)
        cv = cline[sl]
        wv = wline[sl]
        chalf[sl] = (cv >> hsh) & ibit
        whalf[sl] = (wv >> hsh) & ibit
        cline[sl] = ((cv >> (hsh + 1)) << hsh) + (cv & lmask)
        wline[sl] = ((wv >> (hsh + 1)) << hsh) + (wv & lmask)

    row_iota = lax.iota(jnp.int32, LANES)
    fone = jnp.full((LANES,), 1.0, jnp.float32)
    izero = jnp.zeros((LANES,), jnp.int32)

    @pl.loop(0, NCH)
    def _(ch):
        p0 = ch * PCHUNK
        cg = pltpu.async_copy(
            cpack_hbm.at[cline.at[pl.ds(p0, PCHUNK)]], cbuf, sem)
        wg = pltpu.async_copy(
            wpack_hbm.at[wline.at[pl.ds(p0, PCHUNK)]], wbuf, sem)
        cg.wait()
        wg.wait()

        for g in range(PCHUNK // GROUP):
            for r in range(GROUP):
                p = g * GROUP + r
                psplat = jnp.full((LANES,), p0 + p, jnp.int32)
                cmask = plsc.load_gather(chalf, [psplat]) > izero
                wmask = plsc.load_gather(whalf, [psplat]) > izero
                acc = None
                for k in range(KCH):
                    clo = cbuf[p, pl.ds(k * LANES, LANES)]
                    chi = cbuf[p, pl.ds(EMBED + k * LANES, LANES)]
                    wlo = wbuf[p, pl.ds(k * LANES, LANES)]
                    whi = wbuf[p, pl.ds(EMBED + k * LANES, LANES)]
                    cv = jnp.where(cmask, chi, clo)
                    wv = jnp.where(wmask, whi, wlo)
                    prod = cv * wv
                    acc = prod if acc is None else acc + prod
                accbuf[r, :] = acc
            tot = None
            for j in range(LANES):
                col = plsc.load_gather(
                    accbuf, [row_iota, jnp.full((LANES,), j, jnp.int32)])
                tot = col if tot is None else tot + col
            outv[pl.ds(p0 + g * GROUP, GROUP)] = fone / (fone + jnp.exp(-tot))

    pltpu.sync_copy(outv, out_hbm.at[pl.ds(base, BPW)])


def kernel(c, w, c_table, w_table):
    cpack, wpack = _pack(c_table.T, w_table.T)
    return _sgns_sc(c, w, cpack, wpack)


# speedup vs baseline: 3.1838x; 1.1768x over previous
import dataclasses
import functools

import jax
import jax.numpy as jnp
from jax import lax
from jax.experimental import pallas as pl
from jax.experimental.pallas import tpu as pltpu
from jax.experimental.pallas import tpu_sc as plsc

EMBED = 64
LINE = 2 * EMBED
VOCAB = 1000000
RBLK = 8192
NPACK = (VOCAB + RBLK - 1) // RBLK
NLINES = NPACK * (RBLK // 2)
LANES = 16
NCORE = 2
NSUB = 16
NWORK = NCORE * NSUB
BATCH = 16384
BPW = BATCH // NWORK
PCHUNK = 128
NCH = BPW // PCHUNK
GROUP = LANES
KCH = EMBED // LANES

_cp = pltpu.CompilerParams(use_tc_tiling_on_sc=False)
if "needs_layout_passes" in pltpu.CompilerParams.__dataclass_fields__:
    _cp = dataclasses.replace(_cp, needs_layout_passes=False)


def _pack_body(ct_ref, wt_ref, cp_ref, wp_ref):
    cp_ref[:, 0:EMBED] = ct_ref[:, 0:RBLK // 2].T
    cp_ref[:, EMBED:LINE] = ct_ref[:, RBLK // 2:RBLK].T
    wp_ref[:, 0:EMBED] = wt_ref[:, 0:RBLK // 2].T
    wp_ref[:, EMBED:LINE] = wt_ref[:, RBLK // 2:RBLK].T


def _pack(ctT, wtT):
    return pl.pallas_call(
        _pack_body,
        grid=(NPACK,),
        in_specs=[
            pl.BlockSpec((EMBED, RBLK), lambda q: (0, q)),
            pl.BlockSpec((EMBED, RBLK), lambda q: (0, q)),
        ],
        out_specs=[
            pl.BlockSpec((RBLK // 2, LINE), lambda q: (q, 0)),
            pl.BlockSpec((RBLK // 2, LINE), lambda q: (q, 0)),
        ],
        out_shape=[
            jax.ShapeDtypeStruct((NLINES, LINE), jnp.float32),
            jax.ShapeDtypeStruct((NLINES, LINE), jnp.float32),
        ],
        compiler_params=pltpu.CompilerParams(
            dimension_semantics=("parallel",)),
    )(ctT, wtT)


@functools.partial(
    pl.kernel,
    compiler_params=_cp,
    out_type=jax.ShapeDtypeStruct((BATCH,), jnp.float32),
    mesh=plsc.VectorSubcoreMesh(core_axis_name="c", subcore_axis_name="s"),
    scratch_types=[
        pltpu.VMEM((BPW,), jnp.int32),
        pltpu.VMEM((BPW,), jnp.int32),
        pltpu.VMEM((BPW,), jnp.int32),
        pltpu.VMEM((BPW,), jnp.int32),
        pltpu.VMEM((PCHUNK, LINE), jnp.float32),
        pltpu.VMEM((PCHUNK, LINE), jnp.float32),
        pltpu.VMEM((GROUP, LANES), jnp.float32),
        pltpu.VMEM((BPW,), jnp.float32),
        pltpu.SemaphoreType.DMA,
    ],
)
def _sgns_sc(c_hbm, w_hbm, cpack_hbm, wpack_hbm, out_hbm,
             cline, wline, chalf, whalf, cbuf, wbuf, accbuf, outv, sem):
    wid = lax.axis_index("s") * NCORE + lax.axis_index("c")
    base = wid * BPW

    pltpu.sync_copy(c_hbm.at[pl.ds(base, BPW)], cline)
    pltpu.sync_copy(w_hbm.at[pl.ds(base, BPW)], wline)
    lmask = jnp.full((LANES,), RBLK // 2 - 1, jnp.int32)
    ibit = jnp.full((LANES,), 1, jnp.int32)
    hsh = (RBLK // 2).bit_length() - 1
    for v in range(BPW // LANES):
        sl = pl.ds(v * LANES, LANES)
        cv = cline[sl]
        wv = wline[sl]
        chalf[sl] = (cv >> hsh) & ibit
        whalf[sl] = (wv >> hsh) & ibit
        cline[sl] = ((cv >> (hsh + 1)) << hsh) + (cv & lmask)
        wline[sl] = ((wv >> (hsh + 1)) << hsh) + (wv & lmask)

    row_iota = lax.iota(jnp.int32, LANES)
    fone = jnp.full((LANES,), 1.0, jnp.float32)
    izero = jnp.zeros((LANES,), jnp.int32)

    @pl.loop(0, NCH)
    def _(ch):
        p0 = ch * PCHUNK
        cg = pltpu.async_copy(
            cpack_hbm.at[cline.at[pl.ds(p0, PCHUNK)]], cbuf, sem)
        wg = pltpu.async_copy(
            wpack_hbm.at[wline.at[pl.ds(p0, PCHUNK)]], wbuf, sem)
        cg.wait()
        wg.wait()

        for g in range(PCHUNK // GROUP):
            for r in range(GROUP):
                p = g * GROUP + r
                psplat = jnp.full((LANES,), p0 + p, jnp.int32)
                cmask = plsc.load_gather(chalf, [psplat]) > izero
                wmask = plsc.load_gather(whalf, [psplat]) > izero
                acc = None
                for k in range(KCH):
                    clo = cbuf[p, pl.ds(k * LANES, LANES)]
                    chi = cbuf[p, pl.ds(EMBED + k * LANES, LANES)]
                    wlo = wbuf[p, pl.ds(k * LANES, LANES)]
                    whi = wbuf[p, pl.ds(EMBED + k * LANES, LANES)]
                    cv = jnp.where(cmask, chi, clo)
                    wv = jnp.where(wmask, whi, wlo)
                    prod = cv * wv
                    acc = prod if acc is None else acc + prod
                accbuf[r, :] = acc
            tot = None
            for j in range(LANES):
                col = plsc.load_gather(
                    accbuf, [row_iota, jnp.full((LANES,), j, jnp.int32)])
                tot = col if tot is None else tot + col
            outv[pl.ds(p0 + g * GROUP, GROUP)] = fone / (fone + jnp.exp(-tot))

    pltpu.sync_copy(outv, out_hbm.at[pl.ds(base, BPW)])


def kernel(c, w, c_table, w_table):
    cpack, wpack = _pack(c_table.T, w_table.T)
    return _sgns_sc(c, w, cpack, wpack)


# speedup vs baseline: 3.2233x; 1.0124x over previous
import dataclasses
import functools

import jax
import jax.numpy as jnp
from jax import lax
from jax.experimental import pallas as pl
from jax.experimental.pallas import tpu as pltpu
from jax.experimental.pallas import tpu_sc as plsc

EMBED = 64
LINE = 2 * EMBED
VOCAB = 1000000
RBLK = 16384
NPACK = (VOCAB + RBLK - 1) // RBLK
NLINES = NPACK * (RBLK // 2)
LANES = 16
NCORE = 2
NSUB = 16
NWORK = NCORE * NSUB
BATCH = 16384
BPW = BATCH // NWORK
PCHUNK = 128
NCH = BPW // PCHUNK
GROUP = LANES
KCH = EMBED // LANES

_cp = pltpu.CompilerParams(use_tc_tiling_on_sc=False)
if "needs_layout_passes" in pltpu.CompilerParams.__dataclass_fields__:
    _cp = dataclasses.replace(_cp, needs_layout_passes=False)


def _pack_body(ct_ref, wt_ref, cp_ref, wp_ref):
    cp_ref[:, 0:EMBED] = ct_ref[:, 0:RBLK // 2].T
    cp_ref[:, EMBED:LINE] = ct_ref[:, RBLK // 2:RBLK].T
    wp_ref[:, 0:EMBED] = wt_ref[:, 0:RBLK // 2].T
    wp_ref[:, EMBED:LINE] = wt_ref[:, RBLK // 2:RBLK].T


def _pack(ctT, wtT):
    return pl.pallas_call(
        _pack_body,
        grid=(NPACK,),
        in_specs=[
            pl.BlockSpec((EMBED, RBLK), lambda q: (0, q)),
            pl.BlockSpec((EMBED, RBLK), lambda q: (0, q)),
        ],
        out_specs=[
            pl.BlockSpec((RBLK // 2, LINE), lambda q: (q, 0)),
            pl.BlockSpec((RBLK // 2, LINE), lambda q: (q, 0)),
        ],
        out_shape=[
            jax.ShapeDtypeStruct((NLINES, LINE), jnp.float32),
            jax.ShapeDtypeStruct((NLINES, LINE), jnp.float32),
        ],
        compiler_params=pltpu.CompilerParams(
            dimension_semantics=("parallel",)),
    )(ctT, wtT)


@functools.partial(
    pl.kernel,
    compiler_params=_cp,
    out_type=jax.ShapeDtypeStruct((BATCH,), jnp.float32),
    mesh=plsc.VectorSubcoreMesh(core_axis_name="c", subcore_axis_name="s"),
    scratch_types=[
        pltpu.VMEM((BPW,), jnp.int32),
        pltpu.VMEM((BPW,), jnp.int32),
        pltpu.VMEM((BPW,), jnp.int32),
        pltpu.VMEM((BPW,), jnp.int32),
        pltpu.VMEM((PCHUNK, LINE), jnp.float32),
        pltpu.VMEM((PCHUNK, LINE), jnp.float32),
        pltpu.VMEM((GROUP, LANES), jnp.float32),
        pltpu.VMEM((BPW,), jnp.float32),
        pltpu.SemaphoreType.DMA,
    ],
)
def _sgns_sc(c_hbm, w_hbm, cpack_hbm, wpack_hbm, out_hbm,
             cline, wline, chalf, whalf, cbuf, wbuf, accbuf, outv, sem):
    wid = lax.axis_index("s") * NCORE + lax.axis_index("c")
    base = wid * BPW

    pltpu.sync_copy(c_hbm.at[pl.ds(base, BPW)], cline)
    pltpu.sync_copy(w_hbm.at[pl.ds(base, BPW)], wline)
    lmask = jnp.full((LANES,), RBLK // 2 - 1, jnp.int32)
    ibit = jnp.full((LANES,), 1, jnp.int32)
    hsh = (RBLK // 2).bit_length() - 1
    for v in range(BPW // LANES):
        sl = pl.ds(v * LANES, LANES)
        cv = cline[sl]
        wv = wline[sl]
        chalf[sl] = (cv >> hsh) & ibit
        whalf[sl] = (wv >> hsh) & ibit
        cline[sl] = ((cv >> (hsh + 1)) << hsh) + (cv & lmask)
        wline[sl] = ((wv >> (hsh + 1)) << hsh) + (wv & lmask)

    row_iota = lax.iota(jnp.int32, LANES)
    fone = jnp.full((LANES,), 1.0, jnp.float32)
    izero = jnp.zeros((LANES,), jnp.int32)

    @pl.loop(0, NCH)
    def _(ch):
        p0 = ch * PCHUNK
        cg = pltpu.async_copy(
            cpack_hbm.at[cline.at[pl.ds(p0, PCHUNK)]], cbuf, sem)
        wg = pltpu.async_copy(
            wpack_hbm.at[wline.at[pl.ds(p0, PCHUNK)]], wbuf, sem)
        cg.wait()
        wg.wait()

        for g in range(PCHUNK // GROUP):
            for r in range(GROUP):
                p = g * GROUP + r
                psplat = jnp.full((LANES,), p0 + p, jnp.int32)
                cmask = plsc.load_gather(chalf, [psplat]) > izero
                wmask = plsc.load_gather(whalf, [psplat]) > izero
                acc = None
                for k in range(KCH):
                    clo = cbuf[p, pl.ds(k * LANES, LANES)]
                    chi = cbuf[p, pl.ds(EMBED + k * LANES, LANES)]
                    wlo = wbuf[p, pl.ds(k * LANES, LANES)]
                    whi = wbuf[p, pl.ds(EMBED + k * LANES, LANES)]
                    cv = jnp.where(cmask, chi, clo)
                    wv = jnp.where(wmask, whi, wlo)
                    prod = cv * wv
                    acc = prod if acc is None else acc + prod
                accbuf[r, :] = acc
            tot = None
            for j in range(LANES):
                col = plsc.load_gather(
                    accbuf, [row_iota, jnp.full((LANES,), j, jnp.int32)])
                tot = col if tot is None else tot + col
            outv[pl.ds(p0 + g * GROUP, GROUP)] = fone / (fone + jnp.exp(-tot))

    pltpu.sync_copy(outv, out_hbm.at[pl.ds(base, BPW)])


def kernel(c, w, c_table, w_table):
    cpack, wpack = _pack(c_table.T, w_table.T)
    return _sgns_sc(c, w, cpack, wpack)
